# baseline (speedup 1.0000x reference)
import jax
import jax.numpy as jnp
from jax import lax
from jax.experimental import pallas as pl
from jax.experimental.pallas import tpu as pltpu
from jax.experimental.pallas import tpu_sc as plsc

NUM_CORES = 2
NUM_SUBCORES = 16
LANES = 16
NUM_WORKERS = NUM_CORES * NUM_SUBCORES

BATCH = 16384
EMB_DIM = 64
PAIR_W = 2 * EMB_DIM
B_PER_W = BATCH // NUM_WORKERS
IDX_CHUNK = 128
N_CHUNKS = B_PER_W // IDX_CHUNK
PASS_ROWS = 256
N_PASS = B_PER_W // PASS_ROWS
CHUNKS_PER_PASS = PASS_ROWS // IDX_CHUNK
PASS_GROUPS = PASS_ROWS // LANES


def _bpr_body(bu_hbm, bpi_hbm, bni_hbm, ur_hbm, ir_hbm,
              pos_hbm, neg_hbm,
              idx_u, idx_i, idx_j, half_u, half_i, half_j,
              rows_u, rows_i, rows_j, pos_v, neg_v, sem):
    wid = lax.axis_index("s") * NUM_CORES + lax.axis_index("c")
    base = wid * B_PER_W

    for k in range(N_CHUNKS):
        off = base + k * IDX_CHUNK
        pltpu.sync_copy(bu_hbm.at[pl.ds(off, IDX_CHUNK)], idx_u.at[k])
        pltpu.sync_copy(bpi_hbm.at[pl.ds(off, IDX_CHUNK)], idx_i.at[k])
        pltpu.sync_copy(bni_hbm.at[pl.ds(off, IDX_CHUNK)], idx_j.at[k])
    for k in range(N_CHUNKS):
        for v in range(IDX_CHUNK // LANES):
            sl = pl.ds(v * LANES, LANES)
            for idx, half in ((idx_u, half_u), (idx_i, half_i), (idx_j, half_j)):
                raw = idx[k, sl]
                half[k, sl] = (raw & 1) * EMB_DIM
                idx[k, sl] = raw >> 1

    def pass_body(p, carry):
        for k in range(CHUNKS_PER_PASS):
            kk = p * CHUNKS_PER_PASS + k
            sl = pl.ds(k * IDX_CHUNK, IDX_CHUNK)
            pltpu.async_copy(ur_hbm.at[idx_u.at[kk]], rows_u.at[sl], sem)
            pltpu.async_copy(ir_hbm.at[idx_i.at[kk]], rows_i.at[sl], sem)
            pltpu.async_copy(ir_hbm.at[idx_j.at[kk]], rows_j.at[sl], sem)
        for k in range(CHUNKS_PER_PASS):
            kk = p * CHUNKS_PER_PASS + k
            sl = pl.ds(k * IDX_CHUNK, IDX_CHUNK)
            pltpu.make_async_copy(ur_hbm.at[idx_u.at[kk]],
                                  rows_u.at[sl], sem).wait()
            pltpu.make_async_copy(ir_hbm.at[idx_i.at[kk]],
                                  rows_i.at[sl], sem).wait()
            pltpu.make_async_copy(ir_hbm.at[idx_j.at[kk]],
                                  rows_j.at[sl], sem).wait()

        def group_body(g, inner):
            row = g * LANES + lax.iota(jnp.int32, LANES)
            kk0 = p * CHUNKS_PER_PASS
            gsl = pl.ds(g * LANES, LANES)
            hk = kk0 + g // (IDX_CHUNK // LANES)
            hsl = pl.ds((g % (IDX_CHUNK // LANES)) * LANES, LANES)
            cb_u = half_u[hk, hsl]
            cb_i = half_i[hk, hsl]
            cb_j = half_j[hk, hsl]
            accp = jnp.zeros((LANES,), jnp.float32)
            accn = jnp.zeros((LANES,), jnp.float32)
            for c in range(EMB_DIM):
                u = plsc.load_gather(rows_u, [row, cb_u + c])
                iv = plsc.load_gather(rows_i, [row, cb_i + c])
                jv = plsc.load_gather(rows_j, [row, cb_j + c])
                accp = accp + u * iv
                accn = accn + u * jv
            out = pl.ds(p * PASS_ROWS + g * LANES, LANES)
            pos_v[out] = accp
            neg_v[out] = accn
            return inner

        lax.fori_loop(0, PASS_GROUPS, group_body, 0, unroll=False)
        return carry

    lax.fori_loop(0, N_PASS, pass_body, 0, unroll=False)

    pltpu.sync_copy(pos_v, pos_hbm.at[pl.ds(base, B_PER_W)])
    pltpu.sync_copy(neg_v, neg_hbm.at[pl.ds(base, B_PER_W)])


@jax.jit
def _bpr_scores(batch_user, batch_pos_item, batch_neg_item,
                user_emb, item_emb):
    ur = user_emb.reshape(NUM_USER_PAIRS, PAIR_W)
    ir = item_emb.reshape(NUM_ITEM_PAIRS, PAIR_W)
    mesh = plsc.VectorSubcoreMesh(core_axis_name="c", subcore_axis_name="s",
                                  num_cores=NUM_CORES,
                                  num_subcores=NUM_SUBCORES)
    run = pl.kernel(
        _bpr_body,
        out_type=[jax.ShapeDtypeStruct((BATCH,), jnp.float32),
                  jax.ShapeDtypeStruct((BATCH,), jnp.float32)],
        mesh=mesh,
        compiler_params=pltpu.CompilerParams(needs_layout_passes=False,
                                             use_tc_tiling_on_sc=True),
        scratch_types=[
            pltpu.VMEM((N_CHUNKS, IDX_CHUNK), jnp.int32),
            pltpu.VMEM((N_CHUNKS, IDX_CHUNK), jnp.int32),
            pltpu.VMEM((N_CHUNKS, IDX_CHUNK), jnp.int32),
            pltpu.VMEM((N_CHUNKS, IDX_CHUNK), jnp.int32),
            pltpu.VMEM((N_CHUNKS, IDX_CHUNK), jnp.int32),
            pltpu.VMEM((N_CHUNKS, IDX_CHUNK), jnp.int32),
            pltpu.VMEM((PASS_ROWS, PAIR_W), jnp.float32),
            pltpu.VMEM((PASS_ROWS, PAIR_W), jnp.float32),
            pltpu.VMEM((PASS_ROWS, PAIR_W), jnp.float32),
            pltpu.VMEM((B_PER_W,), jnp.float32),
            pltpu.VMEM((B_PER_W,), jnp.float32),
            pltpu.SemaphoreType.DMA,
        ],
    )
    return run(batch_user, batch_pos_item, batch_neg_item, ur, ir)


NUM_USER_PAIRS = 500000
NUM_ITEM_PAIRS = 500000


def kernel(batch_user, batch_pos_item, batch_neg_item, user_emb, item_emb):
    pos, neg = _bpr_scores(batch_user.astype(jnp.int32),
                           batch_pos_item.astype(jnp.int32),
                           batch_neg_item.astype(jnp.int32),
                           user_emb, item_emb)
    return (pos[:, None], neg[:, None])


# speedup vs baseline: 1.3896x; 1.3896x over previous
import jax
import jax.numpy as jnp
from jax import lax
from jax.experimental import pallas as pl
from jax.experimental.pallas import tpu as pltpu
from jax.experimental.pallas import tpu_sc as plsc

NUM_CORES = 2
NUM_SUBCORES = 16
LANES = 16
NUM_WORKERS = NUM_CORES * NUM_SUBCORES

BATCH = 16384
EMB_DIM = 64
NROW = 1000000
RANGE = 31232
CHUNK = 512
N_CHUNK = RANGE // CHUNK
TAIL0 = NUM_WORKERS * RANGE
TAIL_MAIN = 512
TAIL_PATCH = 128
TAILP0 = NROW - TAIL_PATCH
LIST_CAP = 3 * BATCH + LANES
IDXC = 2048
B_PER_W = BATCH // NUM_WORKERS
RING = 32
GROUPS = B_PER_W // LANES

def _scan_body(bu_hbm, bpi_hbm, bni_hbm, ut_hbm, it_hbm, utail_hbm, itail_hbm,
               du_hbm, di_hbm, dj_hbm,
               lst, bufa, bufb, tailbuf, idxc, stag, ssem, hsem):
    wid = lax.axis_index("s") * NUM_CORES + lax.axis_index("c")
    lo = wid * RANGE
    is_last = wid == NUM_WORKERS - 1
    hi = jnp.where(is_last, NROW, lo + RANGE)

    def build(arr_hbm, tag):
        def chunk_body(ci, cnt):
            pltpu.sync_copy(arr_hbm.at[pl.ds(ci * IDXC, IDXC)], idxc)

            def vec_body(v, cnt):
                r = idxc[pl.ds(v * LANES, LANES)]
                m = (r >= lo) & (r < hi)
                b = ci * IDXC + v * LANES + lax.iota(jnp.int32, LANES)
                entry = ((r - lo) << 16) | (tag << 14) | b
                plsc.store_compressed(lst.at[pl.ds(cnt, LANES)], entry,
                                      mask=m)
                n = plsc.all_reduce_population_count(m)
                return cnt + n[0]

            return lax.fori_loop(0, IDXC // LANES, vec_body, cnt,
                                 unroll=False)

        return chunk_body

    cnt = 0
    for arr, tag in ((bu_hbm, 0), (bpi_hbm, 1), (bni_hbm, 2)):
        cnt = lax.fori_loop(0, BATCH // IDXC, build(arr, tag), cnt,
                            unroll=False)
    nvec = (cnt + LANES - 1) // LANES

    def drain_one():
        pltpu.make_async_copy(du_hbm.at[pl.ds(0, EMB_DIM)],
                              stag.at[0], hsem).wait()

    def process(buf, c0, width, pass_user):
        def vec_body(v, fired):
            e = lst[pl.ds(v * LANES, LANES)]
            lane = v * LANES + lax.iota(jnp.int32, LANES)
            rloc = e >> 16
            tag = (e >> 14) & 3
            m = (rloc >= c0) & (rloc < c0 + width) & (lane < cnt)
            if pass_user:
                m = m & (tag == 0)
            else:
                m = m & (tag > 0)
            n = plsc.all_reduce_population_count(m)[0]
            mi = m.astype(jnp.int32)
            hnum = plsc.cumsum(mi)

            @pl.when(n > 0)
            def _():
                for l in range(LANES):
                    @pl.when(mi[l] != 0)
                    def _():
                        h = fired + hnum[l] - 1
                        slot = h & (RING - 1)

                        @pl.when(h >= RING)
                        def _():
                            drain_one()

                        el = e[l]
                        cl = (el >> 16) - c0
                        b = el & (BATCH - 1)
                        clv = jnp.full((LANES,), cl, jnp.int32)
                        dims0 = lax.iota(jnp.int32, LANES)
                        for v4 in range(EMB_DIM // LANES):
                            g = plsc.load_gather(buf, [dims0 + v4 * LANES,
                                                       clv])
                            stag[slot, pl.ds(v4 * LANES, LANES)] = g
                        dst = pl.ds(b * EMB_DIM, EMB_DIM)
                        src = stag.at[slot]
                        if pass_user:
                            pltpu.async_copy(src, du_hbm.at[dst], hsem)
                        else:
                            tl = el >> 14
                            @pl.when((tl & 3) == 1)
                            def _():
                                pltpu.async_copy(src, di_hbm.at[dst], hsem)
                            @pl.when((tl & 3) == 2)
                            def _():
                                pltpu.async_copy(src, dj_hbm.at[dst], hsem)

            return fired + n

        fired = lax.fori_loop(0, nvec, vec_body, 0, unroll=False)

        def drain_body(d, carry):
            drain_one()
            return carry

        lax.fori_loop(0, jnp.minimum(fired, RING), drain_body, 0,
                      unroll=False)

    def stream_range(tab_hbm, pass_user):
        def start(k, buf):
            pltpu.async_copy(tab_hbm.at[:, pl.ds(lo + k * CHUNK, CHUNK)],
                             buf, ssem)

        def wait(buf):
            pltpu.make_async_copy(tab_hbm.at[:, pl.ds(0, CHUNK)], buf,
                                  ssem).wait()

        start(0, bufa)

        def chunk_body(k, carry):
            @pl.when(k % 2 == 0)
            def _():
                wait(bufa)
                @pl.when(k + 1 < N_CHUNK)
                def _():
                    start(k + 1, bufb)
                process(bufa, k * CHUNK, CHUNK, pass_user)

            @pl.when(k % 2 == 1)
            def _():
                wait(bufb)
                @pl.when(k + 1 < N_CHUNK)
                def _():
                    start(k + 1, bufa)
                process(bufb, k * CHUNK, CHUNK, pass_user)

            return carry

        lax.fori_loop(0, N_CHUNK, chunk_body, 0, unroll=False)

        @pl.when(is_last)
        def _():
            pltpu.sync_copy(tab_hbm.at[:, pl.ds(TAIL0, TAIL_MAIN)], bufa)
            process(bufa, TAIL0 - lo, TAIL_MAIN, pass_user)
            tail = utail_hbm if pass_user else itail_hbm
            pltpu.sync_copy(tail, tailbuf)
            process(tailbuf, TAILP0 - lo, TAIL_PATCH, pass_user)

    stream_range(ut_hbm, True)
    stream_range(it_hbm, False)


def _dot_body(du_hbm, di_hbm, dj_hbm, pos_hbm, neg_hbm,
              ru, ri, rj, accp_s, accn_s, pos_v, neg_v):
    wid = lax.axis_index("s") * NUM_CORES + lax.axis_index("c")
    base = wid * B_PER_W
    nwords = B_PER_W * EMB_DIM
    pltpu.sync_copy(du_hbm.at[pl.ds(base * EMB_DIM, nwords)], ru)
    pltpu.sync_copy(di_hbm.at[pl.ds(base * EMB_DIM, nwords)], ri)
    pltpu.sync_copy(dj_hbm.at[pl.ds(base * EMB_DIM, nwords)], rj)
    lanes_iota = lax.iota(jnp.int32, LANES)

    def group_body(g, carry):
        for row_l in range(LANES):
            off = (g * LANES + row_l) * EMB_DIM
            accp = jnp.zeros((LANES,), jnp.float32)
            accn = jnp.zeros((LANES,), jnp.float32)
            for v in range(EMB_DIM // LANES):
                sl = pl.ds(off + v * LANES, LANES)
                u = ru[sl]
                iv = ri[sl]
                jv = rj[sl]
                accp = accp + u * iv
                accn = accn + u * jv
            accp_s[row_l] = accp
            accn_s[row_l] = accn
        sump = jnp.zeros((LANES,), jnp.float32)
        sumn = jnp.zeros((LANES,), jnp.float32)
        for l in range(LANES):
            col = jnp.full((LANES,), l, jnp.int32)
            sump = sump + plsc.load_gather(accp_s, [lanes_iota, col])
            sumn = sumn + plsc.load_gather(accn_s, [lanes_iota, col])
        out = pl.ds(g * LANES, LANES)
        pos_v[out] = sump
        neg_v[out] = sumn
        return carry

    lax.fori_loop(0, GROUPS, group_body, 0, unroll=False)
    pltpu.sync_copy(pos_v, pos_hbm.at[pl.ds(base, B_PER_W)])
    pltpu.sync_copy(neg_v, neg_hbm.at[pl.ds(base, B_PER_W)])


@jax.jit
def _bpr_scores(batch_user, batch_pos_item, batch_neg_item,
                user_emb_t, item_emb_t, user_tail, item_tail):
    mesh = plsc.VectorSubcoreMesh(core_axis_name="c", subcore_axis_name="s",
                                  num_cores=NUM_CORES,
                                  num_subcores=NUM_SUBCORES)
    cparams = pltpu.CompilerParams(needs_layout_passes=False,
                                   use_tc_tiling_on_sc=True)
    scan = pl.kernel(
        _scan_body,
        out_type=[jax.ShapeDtypeStruct((BATCH * EMB_DIM,), jnp.float32)] * 3,
        mesh=mesh,
        compiler_params=cparams,
        scratch_types=[
            pltpu.VMEM((LIST_CAP,), jnp.int32),
            pltpu.VMEM((EMB_DIM, CHUNK), jnp.float32),
            pltpu.VMEM((EMB_DIM, CHUNK), jnp.float32),
            pltpu.VMEM((EMB_DIM, TAIL_PATCH), jnp.float32),
            pltpu.VMEM((IDXC,), jnp.int32),
            pltpu.VMEM((RING, EMB_DIM), jnp.float32),
            pltpu.SemaphoreType.DMA,
            pltpu.SemaphoreType.DMA,
        ],
    )
    du, di, dj = scan(batch_user, batch_pos_item, batch_neg_item,
                      user_emb_t, item_emb_t, user_tail, item_tail)
    dot = pl.kernel(
        _dot_body,
        out_type=[jax.ShapeDtypeStruct((BATCH,), jnp.float32)] * 2,
        mesh=mesh,
        compiler_params=cparams,
        scratch_types=[
            pltpu.VMEM((B_PER_W * EMB_DIM,), jnp.float32),
            pltpu.VMEM((B_PER_W * EMB_DIM,), jnp.float32),
            pltpu.VMEM((B_PER_W * EMB_DIM,), jnp.float32),
            pltpu.VMEM((LANES, LANES), jnp.float32),
            pltpu.VMEM((LANES, LANES), jnp.float32),
            pltpu.VMEM((B_PER_W,), jnp.float32),
            pltpu.VMEM((B_PER_W,), jnp.float32),
        ],
    )
    return dot(du, di, dj)


def kernel(batch_user, batch_pos_item, batch_neg_item, user_emb, item_emb):
    ut = user_emb.T
    it = item_emb.T
    pos, neg = _bpr_scores(batch_user.astype(jnp.int32),
                           batch_pos_item.astype(jnp.int32),
                           batch_neg_item.astype(jnp.int32),
                           ut, it,
                           ut[:, TAILP0:],
                           it[:, TAILP0:])
    return (pos[:, None], neg[:, None])


# speedup vs baseline: 1.7152x; 1.2343x over previous
import jax
import jax.numpy as jnp
from jax import lax
from jax.experimental import pallas as pl
from jax.experimental.pallas import tpu as pltpu
from jax.experimental.pallas import tpu_sc as plsc

NUM_CORES = 2
NUM_SUBCORES = 16
LANES = 16
NUM_WORKERS = NUM_CORES * NUM_SUBCORES

BATCH = 16384
EMB_DIM = 64
NROW = 1000000
RANGE = 31232
CHUNK = 512
N_CHUNK = RANGE // CHUNK
TAIL0 = NUM_WORKERS * RANGE
TAIL_MAIN = 512
TAIL_PATCH = 128
TAILP0 = NROW - TAIL_PATCH
LIST_CAP = 2 * BATCH + 2 * LANES
IDXC = 2048
B_PER_W = BATCH // NUM_WORKERS
RING = 32
GROUPS = B_PER_W // LANES
SENTINEL = jnp.int32(0x7FFF0000)


def _scan_body(bu_hbm, bpi_hbm, bni_hbm, ut_hbm, it_hbm, utail_hbm, itail_hbm,
               du_hbm, di_hbm, dj_hbm,
               lst, bufa, bufb, tailbuf, idxc, stag, ssem, hsem):
    wid = lax.axis_index("s") * NUM_CORES + lax.axis_index("c")
    lo = wid * RANGE
    is_last = wid == NUM_WORKERS - 1
    hi = jnp.where(is_last, NROW, lo + RANGE)

    def build(arr_hbm, tag):
        def chunk_body(ci, cnt):
            pltpu.sync_copy(arr_hbm.at[pl.ds(ci * IDXC, IDXC)], idxc)

            def vec_body(v, cnt):
                r = idxc[pl.ds(v * LANES, LANES)]
                m = (r >= lo) & (r < hi)
                b = ci * IDXC + v * LANES + lax.iota(jnp.int32, LANES)
                entry = ((r - lo) << 16) | (tag << 14) | b
                plsc.store_compressed(lst.at[pl.ds(cnt, LANES)], entry,
                                      mask=m)
                n = plsc.all_reduce_population_count(m)
                return cnt + n[0]

            return lax.fori_loop(0, IDXC // LANES, vec_body, cnt,
                                 unroll=False)

        return chunk_body

    def build_list(arrs):
        cnt = 0
        for arr, tag in arrs:
            cnt = lax.fori_loop(0, BATCH // IDXC, build(arr, tag), cnt,
                                unroll=False)
        lst[pl.ds(cnt, LANES)] = jnp.full((LANES,), SENTINEL, jnp.int32)
        return cnt

    def drain_one():
        pltpu.make_async_copy(du_hbm.at[pl.ds(0, EMB_DIM)],
                              stag.at[0], hsem).wait()

    def process(buf, c0, width, pass_user, nvec):
        def vec_body(v, fired):
            e = lst[pl.ds(v * LANES, LANES)]
            rloc = e >> 16
            m = (rloc >= c0) & (rloc < c0 + width)
            n = plsc.all_reduce_population_count(m)[0]

            @pl.when(n > 0)
            def _():
                mi = m.astype(jnp.int32)
                hnum = plsc.cumsum(mi)
                for l in range(LANES):
                    @pl.when(mi[l] != 0)
                    def _():
                        h = fired + hnum[l] - 1
                        slot = h & (RING - 1)

                        @pl.when(h >= RING)
                        def _():
                            drain_one()

                        el = e[l]
                        cl = (el >> 16) - c0
                        b = el & (BATCH - 1)
                        clv = jnp.full((LANES,), cl, jnp.int32)
                        dims0 = lax.iota(jnp.int32, LANES)
                        for v4 in range(EMB_DIM // LANES):
                            g = plsc.load_gather(buf, [dims0 + v4 * LANES,
                                                       clv])
                            stag[slot, pl.ds(v4 * LANES, LANES)] = g
                        dst = pl.ds(b * EMB_DIM, EMB_DIM)
                        src = stag.at[slot]
                        if pass_user:
                            pltpu.async_copy(src, du_hbm.at[dst], hsem)
                        else:
                            tl = el >> 14
                            @pl.when((tl & 3) == 1)
                            def _():
                                pltpu.async_copy(src, di_hbm.at[dst], hsem)
                            @pl.when((tl & 3) == 2)
                            def _():
                                pltpu.async_copy(src, dj_hbm.at[dst], hsem)

            return fired + n

        fired = lax.fori_loop(0, nvec, vec_body, 0, unroll=False)

        def drain_body(d, carry):
            drain_one()
            return carry

        lax.fori_loop(0, jnp.minimum(fired, RING), drain_body, 0,
                      unroll=False)

    def stream_range(tab_hbm, tail_hbm, pass_user, nvec):
        def start(k, buf):
            pltpu.async_copy(tab_hbm.at[:, pl.ds(lo + k * CHUNK, CHUNK)],
                             buf, ssem)

        def wait(buf):
            pltpu.make_async_copy(tab_hbm.at[:, pl.ds(0, CHUNK)], buf,
                                  ssem).wait()

        start(0, bufa)

        def chunk_body(k, carry):
            @pl.when(k % 2 == 0)
            def _():
                wait(bufa)
                @pl.when(k + 1 < N_CHUNK)
                def _():
                    start(k + 1, bufb)
                process(bufa, k * CHUNK, CHUNK, pass_user, nvec)

            @pl.when(k % 2 == 1)
            def _():
                wait(bufb)
                @pl.when(k + 1 < N_CHUNK)
                def _():
                    start(k + 1, bufa)
                process(bufb, k * CHUNK, CHUNK, pass_user, nvec)

            return carry

        lax.fori_loop(0, N_CHUNK, chunk_body, 0, unroll=False)

        @pl.when(is_last)
        def _():
            pltpu.sync_copy(tab_hbm.at[:, pl.ds(TAIL0, TAIL_MAIN)], bufa)
            process(bufa, TAIL0 - lo, TAIL_MAIN, pass_user, nvec)
            pltpu.sync_copy(tail_hbm, tailbuf)
            process(tailbuf, TAILP0 - lo, TAIL_PATCH, pass_user, nvec)

    cnt_u = build_list(((bu_hbm, 0),))
    stream_range(ut_hbm, utail_hbm, True, (cnt_u + LANES - 1) // LANES)
    cnt_i = build_list(((bpi_hbm, 1), (bni_hbm, 2)))
    stream_range(it_hbm, itail_hbm, False, (cnt_i + LANES - 1) // LANES)


def _dot_body(du_hbm, di_hbm, dj_hbm, pos_hbm, neg_hbm,
              ru, ri, rj, accp_s, accn_s, pos_v, neg_v):
    wid = lax.axis_index("s") * NUM_CORES + lax.axis_index("c")
    base = wid * B_PER_W
    nwords = B_PER_W * EMB_DIM
    pltpu.sync_copy(du_hbm.at[pl.ds(base * EMB_DIM, nwords)], ru)
    pltpu.sync_copy(di_hbm.at[pl.ds(base * EMB_DIM, nwords)], ri)
    pltpu.sync_copy(dj_hbm.at[pl.ds(base * EMB_DIM, nwords)], rj)
    lanes_iota = lax.iota(jnp.int32, LANES)

    def group_body(g, carry):
        for row_l in range(LANES):
            off = (g * LANES + row_l) * EMB_DIM
            accp = jnp.zeros((LANES,), jnp.float32)
            accn = jnp.zeros((LANES,), jnp.float32)
            for v in range(EMB_DIM // LANES):
                sl = pl.ds(off + v * LANES, LANES)
                u = ru[sl]
                iv = ri[sl]
                jv = rj[sl]
                accp = accp + u * iv
                accn = accn + u * jv
            accp_s[row_l] = accp
            accn_s[row_l] = accn
        sump = jnp.zeros((LANES,), jnp.float32)
        sumn = jnp.zeros((LANES,), jnp.float32)
        for l in range(LANES):
            col = jnp.full((LANES,), l, jnp.int32)
            sump = sump + plsc.load_gather(accp_s, [lanes_iota, col])
            sumn = sumn + plsc.load_gather(accn_s, [lanes_iota, col])
        out = pl.ds(g * LANES, LANES)
        pos_v[out] = sump
        neg_v[out] = sumn
        return carry

    lax.fori_loop(0, GROUPS, group_body, 0, unroll=False)
    pltpu.sync_copy(pos_v, pos_hbm.at[pl.ds(base, B_PER_W)])
    pltpu.sync_copy(neg_v, neg_hbm.at[pl.ds(base, B_PER_W)])


@jax.jit
def _bpr_scores(batch_user, batch_pos_item, batch_neg_item,
                user_emb_t, item_emb_t, user_tail, item_tail):
    mesh = plsc.VectorSubcoreMesh(core_axis_name="c", subcore_axis_name="s",
                                  num_cores=NUM_CORES,
                                  num_subcores=NUM_SUBCORES)
    cparams = pltpu.CompilerParams(needs_layout_passes=False,
                                   use_tc_tiling_on_sc=True)
    scan = pl.kernel(
        _scan_body,
        out_type=[jax.ShapeDtypeStruct((BATCH * EMB_DIM,), jnp.float32)] * 3,
        mesh=mesh,
        compiler_params=cparams,
        scratch_types=[
            pltpu.VMEM((LIST_CAP,), jnp.int32),
            pltpu.VMEM((EMB_DIM, CHUNK), jnp.float32),
            pltpu.VMEM((EMB_DIM, CHUNK), jnp.float32),
            pltpu.VMEM((EMB_DIM, TAIL_PATCH), jnp.float32),
            pltpu.VMEM((IDXC,), jnp.int32),
            pltpu.VMEM((RING, EMB_DIM), jnp.float32),
            pltpu.SemaphoreType.DMA,
            pltpu.SemaphoreType.DMA,
        ],
    )
    du, di, dj = scan(batch_user, batch_pos_item, batch_neg_item,
                      user_emb_t, item_emb_t, user_tail, item_tail)
    dot = pl.kernel(
        _dot_body,
        out_type=[jax.ShapeDtypeStruct((BATCH,), jnp.float32)] * 2,
        mesh=mesh,
        compiler_params=cparams,
        scratch_types=[
            pltpu.VMEM((B_PER_W * EMB_DIM,), jnp.float32),
            pltpu.VMEM((B_PER_W * EMB_DIM,), jnp.float32),
            pltpu.VMEM((B_PER_W * EMB_DIM,), jnp.float32),
            pltpu.VMEM((LANES, LANES), jnp.float32),
            pltpu.VMEM((LANES, LANES), jnp.float32),
            pltpu.VMEM((B_PER_W,), jnp.float32),
            pltpu.VMEM((B_PER_W,), jnp.float32),
        ],
    )
    return dot(du, di, dj)


def kernel(batch_user, batch_pos_item, batch_neg_item, user_emb, item_emb):
    ut = user_emb.T
    it = item_emb.T
    pos, neg = _bpr_scores(batch_user.astype(jnp.int32),
                           batch_pos_item.astype(jnp.int32),
                           batch_neg_item.astype(jnp.int32),
                           ut, it,
                           ut[:, TAILP0:],
                           it[:, TAILP0:])
    return (pos[:, None], neg[:, None])


# speedup vs baseline: 1.7289x; 1.0080x over previous
import jax
import jax.numpy as jnp
from jax import lax
from jax.experimental import pallas as pl
from jax.experimental.pallas import tpu as pltpu
from jax.experimental.pallas import tpu_sc as plsc

NUM_CORES = 2
NUM_SUBCORES = 16
LANES = 16
NUM_WORKERS = NUM_CORES * NUM_SUBCORES

BATCH = 16384
EMB_DIM = 64
NROW = 1000000
RANGE = 31232
CHUNK = 512
N_CHUNK = RANGE // CHUNK
TAIL0 = NUM_WORKERS * RANGE
TAIL_MAIN = 512
TAIL_PATCH = 128
TAILP0 = NROW - TAIL_PATCH
LIST_CAP = 2 * BATCH + 2 * LANES
IDXC = 2048
B_PER_W = BATCH // NUM_WORKERS
RING = 32
GROUPS = B_PER_W // LANES
SENTINEL = jnp.int32(0x7FFF0000)


def _scan_body(bu_hbm, bpi_hbm, bni_hbm, ut_hbm, it_hbm, utail_hbm, itail_hbm,
               du_hbm, di_hbm, dj_hbm,
               lst, bufa, bufb, tailbuf, idxc, stag, ssem, hsem):
    wid = lax.axis_index("s") * NUM_CORES + lax.axis_index("c")
    lo = wid * RANGE
    is_last = wid == NUM_WORKERS - 1
    hi = jnp.where(is_last, NROW, lo + RANGE)

    def build(arr_hbm, tag):
        def chunk_body(ci, cnt):
            pltpu.sync_copy(arr_hbm.at[pl.ds(ci * IDXC, IDXC)], idxc)

            def vec_body(v, cnt):
                r = idxc[pl.ds(v * LANES, LANES)]
                m = (r >= lo) & (r < hi)
                b = ci * IDXC + v * LANES + lax.iota(jnp.int32, LANES)
                entry = ((r - lo) << 16) | (tag << 14) | b
                plsc.store_compressed(lst.at[pl.ds(cnt, LANES)], entry,
                                      mask=m)
                n = plsc.all_reduce_population_count(m)
                return cnt + n[0]

            return lax.fori_loop(0, IDXC // LANES, vec_body, cnt,
                                 unroll=False)

        return chunk_body

    def build_list(arrs):
        cnt = 0
        for arr, tag in arrs:
            cnt = lax.fori_loop(0, BATCH // IDXC, build(arr, tag), cnt,
                                unroll=False)
        lst[pl.ds(cnt, LANES)] = jnp.full((LANES,), SENTINEL, jnp.int32)
        return cnt

    def drain_one():
        pltpu.make_async_copy(du_hbm.at[pl.ds(0, EMB_DIM)],
                              stag.at[0], hsem).wait()

    def process(buf, c0, width, pass_user, nvec, fired0):
        def vec_body(v, fired):
            e = lst[pl.ds(v * LANES, LANES)]
            rloc = e >> 16
            m = (rloc >= c0) & (rloc < c0 + width)
            n = plsc.all_reduce_population_count(m)[0]

            @pl.when(n > 0)
            def _():
                mi = m.astype(jnp.int32)
                hnum = plsc.cumsum(mi)
                for l in range(LANES):
                    @pl.when(mi[l] != 0)
                    def _():
                        h = fired + hnum[l] - 1
                        slot = h & (RING - 1)

                        @pl.when(h >= RING)
                        def _():
                            drain_one()

                        el = e[l]
                        cl = (el >> 16) - c0
                        b = el & (BATCH - 1)
                        clv = jnp.full((LANES,), cl, jnp.int32)
                        dims0 = lax.iota(jnp.int32, LANES)
                        for v4 in range(EMB_DIM // LANES):
                            g = plsc.load_gather(buf, [dims0 + v4 * LANES,
                                                       clv])
                            stag[slot, pl.ds(v4 * LANES, LANES)] = g
                        dst = pl.ds(b * EMB_DIM, EMB_DIM)
                        src = stag.at[slot]
                        if pass_user:
                            pltpu.async_copy(src, du_hbm.at[dst], hsem)
                        else:
                            tl = el >> 14
                            @pl.when((tl & 3) == 1)
                            def _():
                                pltpu.async_copy(src, di_hbm.at[dst], hsem)
                            @pl.when((tl & 3) == 2)
                            def _():
                                pltpu.async_copy(src, dj_hbm.at[dst], hsem)

            return fired + n

        return lax.fori_loop(0, nvec, vec_body, fired0, unroll=False)

    def stream_range(tab_hbm, tail_hbm, pass_user, nvec):
        def start(k, buf):
            pltpu.async_copy(tab_hbm.at[:, pl.ds(lo + k * CHUNK, CHUNK)],
                             buf, ssem)

        def start_if(k, buf):
            @pl.when(k < N_CHUNK)
            def _():
                start(k, buf)

        def wait(buf):
            pltpu.make_async_copy(tab_hbm.at[:, pl.ds(0, CHUNK)], buf,
                                  ssem).wait()

        start(0, bufa)
        start(1, bufb)

        def pair_body(p, fired):
            k = p * 2
            wait(bufa)
            fired = process(bufa, k * CHUNK, CHUNK, pass_user, nvec, fired)
            start_if(k + 2, bufa)
            wait(bufb)
            fired = process(bufb, (k + 1) * CHUNK, CHUNK, pass_user, nvec,
                            fired)
            start_if(k + 3, bufb)
            return fired

        fired = lax.fori_loop(0, N_CHUNK // 2, pair_body, 0, unroll=False)
        wait(bufa)
        fired = process(bufa, (N_CHUNK - 1) * CHUNK, CHUNK, pass_user, nvec,
                        fired)

        def drain_n(n):
            def drain_body(d, carry):
                drain_one()
                return carry

            lax.fori_loop(0, jnp.minimum(n, RING), drain_body, 0,
                          unroll=False)

        @pl.when(is_last)
        def _():
            pltpu.sync_copy(tab_hbm.at[:, pl.ds(TAIL0, TAIL_MAIN)], bufa)
            f2 = process(bufa, TAIL0 - lo, TAIL_MAIN, pass_user, nvec, fired)
            pltpu.sync_copy(tail_hbm, tailbuf)
            f3 = process(tailbuf, TAILP0 - lo, TAIL_PATCH, pass_user, nvec,
                         f2)
            drain_n(f3)

        @pl.when(jnp.logical_not(is_last))
        def _():
            drain_n(fired)

    cnt_u = build_list(((bu_hbm, 0),))
    stream_range(ut_hbm, utail_hbm, True, (cnt_u + LANES - 1) // LANES)
    cnt_i = build_list(((bpi_hbm, 1), (bni_hbm, 2)))
    stream_range(it_hbm, itail_hbm, False, (cnt_i + LANES - 1) // LANES)


def _dot_body(du_hbm, di_hbm, dj_hbm, pos_hbm, neg_hbm,
              ru, ri, rj, accp_s, accn_s, pos_v, neg_v):
    wid = lax.axis_index("s") * NUM_CORES + lax.axis_index("c")
    base = wid * B_PER_W
    nwords = B_PER_W * EMB_DIM
    pltpu.sync_copy(du_hbm.at[pl.ds(base * EMB_DIM, nwords)], ru)
    pltpu.sync_copy(di_hbm.at[pl.ds(base * EMB_DIM, nwords)], ri)
    pltpu.sync_copy(dj_hbm.at[pl.ds(base * EMB_DIM, nwords)], rj)
    lanes_iota = lax.iota(jnp.int32, LANES)

    def group_body(g, carry):
        for row_l in range(LANES):
            off = (g * LANES + row_l) * EMB_DIM
            accp = jnp.zeros((LANES,), jnp.float32)
            accn = jnp.zeros((LANES,), jnp.float32)
            for v in range(EMB_DIM // LANES):
                sl = pl.ds(off + v * LANES, LANES)
                u = ru[sl]
                iv = ri[sl]
                jv = rj[sl]
                accp = accp + u * iv
                accn = accn + u * jv
            accp_s[row_l] = accp
            accn_s[row_l] = accn
        sump = jnp.zeros((LANES,), jnp.float32)
        sumn = jnp.zeros((LANES,), jnp.float32)
        for l in range(LANES):
            col = jnp.full((LANES,), l, jnp.int32)
            sump = sump + plsc.load_gather(accp_s, [lanes_iota, col])
            sumn = sumn + plsc.load_gather(accn_s, [lanes_iota, col])
        out = pl.ds(g * LANES, LANES)
        pos_v[out] = sump
        neg_v[out] = sumn
        return carry

    lax.fori_loop(0, GROUPS, group_body, 0, unroll=False)
    pltpu.sync_copy(pos_v, pos_hbm.at[pl.ds(base, B_PER_W)])
    pltpu.sync_copy(neg_v, neg_hbm.at[pl.ds(base, B_PER_W)])


@jax.jit
def _bpr_scores(batch_user, batch_pos_item, batch_neg_item,
                user_emb_t, item_emb_t, user_tail, item_tail):
    mesh = plsc.VectorSubcoreMesh(core_axis_name="c", subcore_axis_name="s",
                                  num_cores=NUM_CORES,
                                  num_subcores=NUM_SUBCORES)
    cparams = pltpu.CompilerParams(needs_layout_passes=False,
                                   use_tc_tiling_on_sc=True)
    scan = pl.kernel(
        _scan_body,
        out_type=[jax.ShapeDtypeStruct((BATCH * EMB_DIM,), jnp.float32)] * 3,
        mesh=mesh,
        compiler_params=cparams,
        scratch_types=[
            pltpu.VMEM((LIST_CAP,), jnp.int32),
            pltpu.VMEM((EMB_DIM, CHUNK), jnp.float32),
            pltpu.VMEM((EMB_DIM, CHUNK), jnp.float32),
            pltpu.VMEM((EMB_DIM, TAIL_PATCH), jnp.float32),
            pltpu.VMEM((IDXC,), jnp.int32),
            pltpu.VMEM((RING, EMB_DIM), jnp.float32),
            pltpu.SemaphoreType.DMA,
            pltpu.SemaphoreType.DMA,
        ],
    )
    du, di, dj = scan(batch_user, batch_pos_item, batch_neg_item,
                      user_emb_t, item_emb_t, user_tail, item_tail)
    dot = pl.kernel(
        _dot_body,
        out_type=[jax.ShapeDtypeStruct((BATCH,), jnp.float32)] * 2,
        mesh=mesh,
        compiler_params=cparams,
        scratch_types=[
            pltpu.VMEM((B_PER_W * EMB_DIM,), jnp.float32),
            pltpu.VMEM((B_PER_W * EMB_DIM,), jnp.float32),
            pltpu.VMEM((B_PER_W * EMB_DIM,), jnp.float32),
            pltpu.VMEM((LANES, LANES), jnp.float32),
            pltpu.VMEM((LANES, LANES), jnp.float32),
            pltpu.VMEM((B_PER_W,), jnp.float32),
            pltpu.VMEM((B_PER_W,), jnp.float32),
        ],
    )
    return dot(du, di, dj)


def kernel(batch_user, batch_pos_item, batch_neg_item, user_emb, item_emb):
    ut = user_emb.T
    it = item_emb.T
    pos, neg = _bpr_scores(batch_user.astype(jnp.int32),
                           batch_pos_item.astype(jnp.int32),
                           batch_neg_item.astype(jnp.int32),
                           ut, it,
                           ut[:, TAILP0:],
                           it[:, TAILP0:])
    return (pos[:, None], neg[:, None])
